# baseline (speedup 1.0000x reference)
import functools

import jax
import jax.numpy as jnp
from jax import lax
from jax.experimental import pallas as pl
from jax.experimental.pallas import tpu as pltpu
from jax.experimental.pallas import tpu_sc as plsc

B = 2
G = 10000
M = 2500
E = 320000
D = 128

NC = 2
NS = 16
NW = NC * NS
EPW = E // NW
CH = 125
NCHUNK = EPW // CH
CW = 16
MPAD = 2560
RPS = MPAD // NS


def _sc_scatter(grid, src3, dst3):
  mesh = plsc.VectorSubcoreMesh(core_axis_name="c", subcore_axis_name="s")

  @functools.partial(
      pl.kernel,
      mesh=mesh,
      out_type=(
          jax.ShapeDtypeStruct((NC, B, MPAD, D), jnp.float32),
          jax.ShapeDtypeStruct((NC, MPAD, CW), jnp.float32),
      ),
      compiler_params=pltpu.CompilerParams(use_tc_tiling_on_sc=False),
      scratch_types=[
          pltpu.VMEM((NCHUNK, CH), jnp.int32),
          pltpu.VMEM((NCHUNK, CH), jnp.int32),
          pltpu.VMEM((CH, D), jnp.float32),
          pltpu.VMEM((CH, D), jnp.float32),
          pltpu.VMEM((CH, D), jnp.float32),
          pltpu.VMEM((CH, D), jnp.float32),
          pltpu.VMEM((CH, CW), jnp.float32),
          pltpu.VMEM_SHARED((MPAD, D), jnp.float32),
          pltpu.VMEM_SHARED((MPAD, D), jnp.float32),
          pltpu.VMEM_SHARED((MPAD, CW), jnp.float32),
          pltpu.SemaphoreType.DMA,
          pltpu.SemaphoreType.DMA,
      ],
  )
  def k(grid_hbm, src_hbm, dst_hbm, feat_hbm, cnt_hbm,
        src_v, dst_v, a0, a1, b0, b1, ones_v,
        accA, accB, accC, gsem0, gsem1):
    grid0_hbm = grid_hbm.at[0]
    grid1_hbm = grid_hbm.at[1]
    c = lax.axis_index("c")
    s = lax.axis_index("s")
    w = c * NS + s

    pltpu.sync_copy(src_hbm.at[w], src_v)
    pltpu.sync_copy(dst_hbm.at[w], dst_v)

    def zrow(r, carry):
      def zcol(kk, inner):
        a0[r, pl.ds(kk * 16, 16)] = jnp.zeros((16,), jnp.float32)
        return inner
      return lax.fori_loop(0, D // 16, zcol, carry)
    lax.fori_loop(0, CH, zrow, 0)

    def zcrow(r, carry):
      ones_v[r, :] = jnp.zeros((CW,), jnp.float32)
      return carry
    lax.fori_loop(0, CH, zcrow, 0)

    rem = RPS - CH
    base = s * RPS
    pltpu.sync_copy(a0, accA.at[pl.ds(base, CH)])
    pltpu.sync_copy(a0.at[pl.ds(0, rem)], accA.at[pl.ds(base + CH, rem)])
    pltpu.sync_copy(a0, accB.at[pl.ds(base, CH)])
    pltpu.sync_copy(a0.at[pl.ds(0, rem)], accB.at[pl.ds(base + CH, rem)])
    pltpu.sync_copy(ones_v, accC.at[pl.ds(base, CH)])
    pltpu.sync_copy(ones_v.at[pl.ds(0, rem)], accC.at[pl.ds(base + CH, rem)])

    onehot = jnp.where(lax.iota(jnp.int32, CW) == 0, 1.0, 0.0).astype(jnp.float32)
    def orow(r, carry):
      ones_v[r, :] = onehot
      return carry
    lax.fori_loop(0, CH, orow, 0)

    plsc.subcore_barrier()

    def gather_start(j, bufa, bufb, sem):
      pltpu.async_copy(grid0_hbm.at[src_v.at[j]], bufa, sem)
      pltpu.async_copy(grid1_hbm.at[src_v.at[j]], bufb, sem)

    def gather_wait(j, bufa, bufb, sem):
      pltpu.make_async_copy(grid0_hbm.at[src_v.at[j]], bufa, sem).wait()
      pltpu.make_async_copy(grid1_hbm.at[src_v.at[j]], bufb, sem).wait()

    def scatter_add(j, bufa, bufb):
      idx = dst_v.at[j]
      pltpu.sync_copy(bufa, accA.at[idx], add=True)
      pltpu.sync_copy(bufb, accB.at[idx], add=True)
      pltpu.sync_copy(ones_v, accC.at[idx], add=True)

    gather_start(0, a0, b0, gsem0)

    def body(i, carry):
      j = i * 2
      gather_start(j + 1, a1, b1, gsem1)
      gather_wait(j, a0, b0, gsem0)
      scatter_add(j, a0, b0)
      gather_start(j + 2, a0, b0, gsem0)
      gather_wait(j + 1, a1, b1, gsem1)
      scatter_add(j + 1, a1, b1)
      return carry
    lax.fori_loop(0, NCHUNK // 2 - 1, body, 0)

    j = NCHUNK - 2
    gather_start(j + 1, a1, b1, gsem1)
    gather_wait(j, a0, b0, gsem0)
    scatter_add(j, a0, b0)
    gather_wait(j + 1, a1, b1, gsem1)
    scatter_add(j + 1, a1, b1)

    plsc.subcore_barrier()
    pltpu.sync_copy(accA.at[pl.ds(base, CH)], a0)
    pltpu.sync_copy(a0, feat_hbm.at[c, 0, pl.ds(base, CH)])
    pltpu.sync_copy(accA.at[pl.ds(base + CH, rem)], a1.at[pl.ds(0, rem)])
    pltpu.sync_copy(a1.at[pl.ds(0, rem)], feat_hbm.at[c, 0, pl.ds(base + CH, rem)])
    pltpu.sync_copy(accB.at[pl.ds(base, CH)], b0)
    pltpu.sync_copy(b0, feat_hbm.at[c, 1, pl.ds(base, CH)])
    pltpu.sync_copy(accB.at[pl.ds(base + CH, rem)], b1.at[pl.ds(0, rem)])
    pltpu.sync_copy(b1.at[pl.ds(0, rem)], feat_hbm.at[c, 1, pl.ds(base + CH, rem)])
    pltpu.sync_copy(accC.at[pl.ds(base, CH)], ones_v)
    pltpu.sync_copy(ones_v, cnt_hbm.at[c, pl.ds(base, CH)])
    pltpu.sync_copy(accC.at[pl.ds(base + CH, rem)], ones_v.at[pl.ds(0, rem)])
    pltpu.sync_copy(ones_v.at[pl.ds(0, rem)], cnt_hbm.at[c, pl.ds(base + CH, rem)])

  return k(grid, src3, dst3)


def _combine(feat, cnt):
  def body(feat_ref, cnt_ref, out_ref):
    count = jnp.maximum(cnt_ref[0, :M, 0:1] + cnt_ref[1, :M, 0:1], 1.0)
    out_ref[0] = (feat_ref[0, 0, :M] + feat_ref[1, 0, :M]) / count
    out_ref[1] = (feat_ref[0, 1, :M] + feat_ref[1, 1, :M]) / count

  return pl.pallas_call(
      body,
      out_shape=jax.ShapeDtypeStruct((B, M, D), jnp.float32),
  )(feat, cnt)


def kernel(grid_node_features, edge_index):
  src = edge_index[:, 0].astype(jnp.int32).reshape(NW, NCHUNK, CH)
  dst = edge_index[:, 1].astype(jnp.int32).reshape(NW, NCHUNK, CH)
  feat, cnt = _sc_scatter(grid_node_features, src, dst)
  return _combine(feat, cnt)


# speedup vs baseline: 1.0023x; 1.0023x over previous
import functools

import jax
import jax.numpy as jnp
from jax import lax
from jax.experimental import pallas as pl
from jax.experimental.pallas import tpu as pltpu
from jax.experimental.pallas import tpu_sc as plsc

B = 2
G = 10000
M = 2500
E = 320000
D = 128

NC = 2
NS = 16
NW = NC * NS
EPW = E // NW
CH = 100
NCHUNK = EPW // CH
CW = 16
MPAD = 2560
RPS = MPAD // NS


def _sc_scatter(grid, src3, dst3):
  mesh = plsc.VectorSubcoreMesh(core_axis_name="c", subcore_axis_name="s")

  @functools.partial(
      pl.kernel,
      mesh=mesh,
      out_type=(
          jax.ShapeDtypeStruct((NC, B, MPAD, D), jnp.float32),
          jax.ShapeDtypeStruct((NC, MPAD, CW), jnp.float32),
      ),
      compiler_params=pltpu.CompilerParams(
          use_tc_tiling_on_sc=False, needs_layout_passes=False),
      scratch_types=[
          pltpu.VMEM((NCHUNK, CH), jnp.int32),
          pltpu.VMEM((NCHUNK, CH), jnp.int32),
          pltpu.VMEM((CH, D), jnp.float32),
          pltpu.VMEM((CH, D), jnp.float32),
          pltpu.VMEM((CH, D), jnp.float32),
          pltpu.VMEM((CH, D), jnp.float32),
          pltpu.VMEM((CH, CW), jnp.float32),
          pltpu.VMEM((RPS, CW), jnp.float32),
          pltpu.VMEM_SHARED((MPAD, D), jnp.float32),
          pltpu.VMEM_SHARED((MPAD, D), jnp.float32),
          pltpu.VMEM_SHARED((MPAD, CW), jnp.float32),
          pltpu.SemaphoreType.DMA,
          pltpu.SemaphoreType.DMA,
      ],
  )
  def k(grid_hbm, src_hbm, dst_hbm, feat_hbm, cnt_hbm,
        src_v, dst_v, a0, a1, b0, b1, ones_v, cbuf,
        accA, accB, accC, gsem0, gsem1):
    grid0_hbm = grid_hbm.at[0]
    grid1_hbm = grid_hbm.at[1]
    c = lax.axis_index("c")
    s = lax.axis_index("s")
    w = c * NS + s

    pltpu.sync_copy(src_hbm.at[w], src_v)
    pltpu.sync_copy(dst_hbm.at[w], dst_v)

    def zrow(r, carry):
      def zcol(kk, inner):
        a0[r, pl.ds(kk * 16, 16)] = jnp.zeros((16,), jnp.float32)
        return inner
      return lax.fori_loop(0, D // 16, zcol, carry)
    lax.fori_loop(0, CH, zrow, 0)

    def zcrow(r, carry):
      cbuf[r, :] = jnp.zeros((CW,), jnp.float32)
      return carry
    lax.fori_loop(0, RPS, zcrow, 0)

    rem = RPS - CH
    base = s * RPS
    pltpu.sync_copy(a0, accA.at[pl.ds(base, CH)])
    pltpu.sync_copy(a0.at[pl.ds(0, rem)], accA.at[pl.ds(base + CH, rem)])
    pltpu.sync_copy(a0, accB.at[pl.ds(base, CH)])
    pltpu.sync_copy(a0.at[pl.ds(0, rem)], accB.at[pl.ds(base + CH, rem)])
    pltpu.sync_copy(cbuf, accC.at[pl.ds(base, RPS)])

    onehot = jnp.where(lax.iota(jnp.int32, CW) == 0, 1.0, 0.0).astype(jnp.float32)
    def orow(r, carry):
      ones_v[r, :] = onehot
      return carry
    lax.fori_loop(0, CH, orow, 0)

    plsc.subcore_barrier()

    def gather_start(j, bufa, bufb, sem):
      pltpu.async_copy(grid0_hbm.at[src_v.at[j]], bufa, sem)
      pltpu.async_copy(grid1_hbm.at[src_v.at[j]], bufb, sem)

    def gather_wait(j, bufa, bufb, sem):
      pltpu.make_async_copy(grid0_hbm.at[src_v.at[j]], bufa, sem).wait()
      pltpu.make_async_copy(grid1_hbm.at[src_v.at[j]], bufb, sem).wait()

    def scatter_add(j, bufa, bufb):
      idx = dst_v.at[j]
      pltpu.sync_copy(bufa, accA.at[idx], add=True)
      pltpu.sync_copy(bufb, accB.at[idx], add=True)
      pltpu.sync_copy(ones_v, accC.at[idx], add=True)

    gather_start(0, a0, b0, gsem0)

    def body(i, carry):
      j = i * 2
      gather_start(j + 1, a1, b1, gsem1)
      gather_wait(j, a0, b0, gsem0)
      scatter_add(j, a0, b0)
      gather_start(j + 2, a0, b0, gsem0)
      gather_wait(j + 1, a1, b1, gsem1)
      scatter_add(j + 1, a1, b1)
      return carry
    lax.fori_loop(0, NCHUNK // 2 - 1, body, 0)

    j = NCHUNK - 2
    gather_start(j + 1, a1, b1, gsem1)
    gather_wait(j, a0, b0, gsem0)
    scatter_add(j, a0, b0)
    gather_wait(j + 1, a1, b1, gsem1)
    scatter_add(j + 1, a1, b1)

    plsc.subcore_barrier()
    pltpu.sync_copy(accA.at[pl.ds(base, CH)], a0)
    pltpu.sync_copy(a0, feat_hbm.at[c, 0, pl.ds(base, CH)])
    pltpu.sync_copy(accA.at[pl.ds(base + CH, rem)], a1.at[pl.ds(0, rem)])
    pltpu.sync_copy(a1.at[pl.ds(0, rem)], feat_hbm.at[c, 0, pl.ds(base + CH, rem)])
    pltpu.sync_copy(accB.at[pl.ds(base, CH)], b0)
    pltpu.sync_copy(b0, feat_hbm.at[c, 1, pl.ds(base, CH)])
    pltpu.sync_copy(accB.at[pl.ds(base + CH, rem)], b1.at[pl.ds(0, rem)])
    pltpu.sync_copy(b1.at[pl.ds(0, rem)], feat_hbm.at[c, 1, pl.ds(base + CH, rem)])
    pltpu.sync_copy(accC.at[pl.ds(base, RPS)], cbuf)
    pltpu.sync_copy(cbuf, cnt_hbm.at[c, pl.ds(base, RPS)])

  return k(grid, src3, dst3)


def _combine(feat, cnt):
  def body(feat_ref, cnt_ref, out_ref):
    count = jnp.maximum(cnt_ref[0, :M, 0:1] + cnt_ref[1, :M, 0:1], 1.0)
    out_ref[0] = (feat_ref[0, 0, :M] + feat_ref[1, 0, :M]) / count
    out_ref[1] = (feat_ref[0, 1, :M] + feat_ref[1, 1, :M]) / count

  return pl.pallas_call(
      body,
      out_shape=jax.ShapeDtypeStruct((B, M, D), jnp.float32),
  )(feat, cnt)


def kernel(grid_node_features, edge_index):
  src = edge_index[:, 0].astype(jnp.int32).reshape(NW, NCHUNK, CH)
  dst = edge_index[:, 1].astype(jnp.int32).reshape(NW, NCHUNK, CH)
  feat, cnt = _sc_scatter(grid_node_features, src, dst)
  return _combine(feat, cnt)


# speedup vs baseline: 1.0173x; 1.0149x over previous
import functools

import jax
import jax.numpy as jnp
from jax import lax
from jax.experimental import pallas as pl
from jax.experimental.pallas import tpu as pltpu
from jax.experimental.pallas import tpu_sc as plsc

B = 2
G = 10000
M = 2500
E = 320000
D = 128

NC = 2
NS = 16
NW = NC * NS
EPW = E // NW
CH = 100
NCHUNK = EPW // CH
CW = 16
MPAD = 2560
RPS = MPAD // NS


def _sc_scatter(grid, src3, dst3):
  mesh = plsc.VectorSubcoreMesh(core_axis_name="c", subcore_axis_name="s")

  @functools.partial(
      pl.kernel,
      mesh=mesh,
      out_type=(
          jax.ShapeDtypeStruct((NC, B, MPAD, D), jnp.float32),
          jax.ShapeDtypeStruct((NC, NS, MPAD), jnp.float32),
      ),
      compiler_params=pltpu.CompilerParams(
          use_tc_tiling_on_sc=False, needs_layout_passes=False),
      scratch_types=[
          pltpu.VMEM((NCHUNK, CH), jnp.int32),
          pltpu.VMEM((NCHUNK, CH), jnp.int32),
          pltpu.VMEM((CH, D), jnp.float32),
          pltpu.VMEM((CH, D), jnp.float32),
          pltpu.VMEM((CH, D), jnp.float32),
          pltpu.VMEM((CH, D), jnp.float32),
          pltpu.VMEM((MPAD,), jnp.float32),
          pltpu.VMEM_SHARED((MPAD, D), jnp.float32),
          pltpu.VMEM_SHARED((MPAD, D), jnp.float32),
          pltpu.SemaphoreType.DMA,
          pltpu.SemaphoreType.DMA,
      ],
  )
  def k(grid_hbm, src_hbm, dst_hbm, feat_hbm, cnt_hbm,
        src_v, dst_v, a0, a1, b0, b1, cnt_v,
        accA, accB, gsem0, gsem1):
    grid0_hbm = grid_hbm.at[0]
    grid1_hbm = grid_hbm.at[1]
    c = lax.axis_index("c")
    s = lax.axis_index("s")
    w = c * NS + s

    pltpu.sync_copy(src_hbm.at[w], src_v)
    pltpu.sync_copy(dst_hbm.at[w], dst_v)

    def zrow(r, carry):
      def zcol(kk, inner):
        a0[r, pl.ds(kk * 16, 16)] = jnp.zeros((16,), jnp.float32)
        return inner
      return lax.fori_loop(0, D // 16, zcol, carry)
    lax.fori_loop(0, CH, zrow, 0)

    lanes = lax.iota(jnp.int32, 16)
    onesf = jnp.ones((16,), jnp.float32)
    def zcnt(t, carry):
      cnt_v[pl.ds(t * 16, 16)] = jnp.zeros((16,), jnp.float32)
      return carry
    lax.fori_loop(0, MPAD // 16, zcnt, 0)

    rem = RPS - CH
    base = s * RPS
    pltpu.sync_copy(a0, accA.at[pl.ds(base, CH)])
    pltpu.sync_copy(a0.at[pl.ds(0, rem)], accA.at[pl.ds(base + CH, rem)])
    pltpu.sync_copy(a0, accB.at[pl.ds(base, CH)])
    pltpu.sync_copy(a0.at[pl.ds(0, rem)], accB.at[pl.ds(base + CH, rem)])

    plsc.subcore_barrier()

    def gather_start(j, bufa, bufb, sem):
      pltpu.async_copy(grid0_hbm.at[src_v.at[j]], bufa, sem)
      pltpu.async_copy(grid1_hbm.at[src_v.at[j]], bufb, sem)

    def gather_wait(j, bufa, bufb, sem):
      pltpu.make_async_copy(grid0_hbm.at[src_v.at[j]], bufa, sem).wait()
      pltpu.make_async_copy(grid1_hbm.at[src_v.at[j]], bufb, sem).wait()

    def scatter_add(j, bufa, bufb):
      idx = dst_v.at[j]
      pltpu.sync_copy(bufa, accA.at[idx], add=True)
      pltpu.sync_copy(bufb, accB.at[idx], add=True)

    tmask = lanes < (CH - (CH // 16) * 16)

    def count_add(j):
      for kk in range(CH // 16):
        idx = dst_v[j, pl.ds(kk * 16, 16)]
        plsc.addupdate_scatter(cnt_v, [idx], onesf)
      jfull = jnp.full((16,), j, jnp.int32)
      idxt = plsc.load_gather(dst_v, [jfull, (CH // 16) * 16 + lanes], mask=tmask)
      plsc.addupdate_scatter(cnt_v, [idxt], onesf, mask=tmask)

    gather_start(0, a0, b0, gsem0)

    def body(i, carry):
      j = i * 2
      gather_start(j + 1, a1, b1, gsem1)
      gather_wait(j, a0, b0, gsem0)
      scatter_add(j, a0, b0)
      count_add(j)
      gather_start(j + 2, a0, b0, gsem0)
      gather_wait(j + 1, a1, b1, gsem1)
      scatter_add(j + 1, a1, b1)
      count_add(j + 1)
      return carry
    lax.fori_loop(0, NCHUNK // 2 - 1, body, 0)

    j = NCHUNK - 2
    gather_start(j + 1, a1, b1, gsem1)
    gather_wait(j, a0, b0, gsem0)
    scatter_add(j, a0, b0)
    count_add(j)
    gather_wait(j + 1, a1, b1, gsem1)
    scatter_add(j + 1, a1, b1)
    count_add(j + 1)

    pltpu.sync_copy(cnt_v, cnt_hbm.at[c, s])

    plsc.subcore_barrier()
    pltpu.sync_copy(accA.at[pl.ds(base, CH)], a0)
    pltpu.sync_copy(a0, feat_hbm.at[c, 0, pl.ds(base, CH)])
    pltpu.sync_copy(accA.at[pl.ds(base + CH, rem)], a1.at[pl.ds(0, rem)])
    pltpu.sync_copy(a1.at[pl.ds(0, rem)], feat_hbm.at[c, 0, pl.ds(base + CH, rem)])
    pltpu.sync_copy(accB.at[pl.ds(base, CH)], b0)
    pltpu.sync_copy(b0, feat_hbm.at[c, 1, pl.ds(base, CH)])
    pltpu.sync_copy(accB.at[pl.ds(base + CH, rem)], b1.at[pl.ds(0, rem)])
    pltpu.sync_copy(b1.at[pl.ds(0, rem)], feat_hbm.at[c, 1, pl.ds(base + CH, rem)])

  return k(grid, src3, dst3)


def _combine(feat, count):
  def body(feat_ref, cnt_ref, out_ref):
    count = cnt_ref[...]
    out_ref[0] = (feat_ref[0, 0, :M] + feat_ref[1, 0, :M]) / count
    out_ref[1] = (feat_ref[0, 1, :M] + feat_ref[1, 1, :M]) / count

  return pl.pallas_call(
      body,
      out_shape=jax.ShapeDtypeStruct((B, M, D), jnp.float32),
  )(feat, count)


def kernel(grid_node_features, edge_index):
  src = edge_index[:, 0].astype(jnp.int32).reshape(NW, NCHUNK, CH)
  dst = edge_index[:, 1].astype(jnp.int32).reshape(NW, NCHUNK, CH)
  feat, cnt = _sc_scatter(grid_node_features, src, dst)
  count = jnp.maximum(cnt.sum(axis=(0, 1))[:M, None], 1.0)
  return _combine(feat, count)
